# Initial kernel scaffold; baseline (speedup 1.0000x reference)
import jax
import jax.numpy as jnp
from jax.experimental import pallas as pl


def kernel(predictions, targets):
    raise NotImplementedError("write your pallas kernel here")


# speedup vs baseline: 162.0074x; 162.0074x over previous
import jax
import jax.numpy as jnp
from jax import lax
from jax.experimental import pallas as pl
from jax.experimental.pallas import tpu as pltpu

_NUM_CLASSES = 80
_CONF = 0.7
_NMS_T = 0.45
_B, _N, _M = 4, 5000, 50
_NP = 5120
_ROWS = _NP // 128


def _prep_body(p_ref, t_ref,
               key_ref, nbx1_ref, nby1_ref, nbx2_ref, nby2_ref,
               x1_ref, y1_ref, x2_ref, y2_ref, cls_ref, sco_ref,
               tx1_ref, ty1_ref, tx2_ref, ty2_ref, tlab_ref, tsco_ref,
               tmsk_ref):
    x = p_ref[0]
    w = x[2:3, :]
    h = x[3:4, :]
    x1 = x[0:1, :] - w * 0.5
    y1 = x[1:2, :] - h * 0.5
    x2 = x1 + w
    y2 = y1 + h
    cls = x[5:5 + _NUM_CLASSES, :]
    m = jnp.max(cls, axis=0, keepdims=True)
    io = lax.broadcasted_iota(jnp.int32, cls.shape, 0)
    am = jnp.min(jnp.where(cls == m, io, _NUM_CLASSES), axis=0, keepdims=True)
    clsf = am.astype(jnp.float32)
    score = x[4:5, :] * m
    valid = score >= _CONF
    key_ref[0] = jnp.where(valid, -score, 1.0)
    off = clsf * 8192.0
    nbx1_ref[0] = x1 + off
    nby1_ref[0] = y1 + off
    nbx2_ref[0] = x2 + off
    nby2_ref[0] = y2 + off
    x1_ref[0] = x1
    y1_ref[0] = y1
    x2_ref[0] = x2
    y2_ref[0] = y2
    cls_ref[0] = clsf
    sco_ref[0] = score

    t = t_ref[0]
    labi = t[0:1, :].astype(jnp.int32)
    tw = t[3:4, :]
    th = t[4:5, :]
    tx1 = t[1:2, :] - tw * 0.5
    ty1 = t[2:3, :] - th * 0.5
    length = jnp.sum((labi > 0).astype(jnp.int32), axis=1, keepdims=True)
    tm = lax.broadcasted_iota(jnp.int32, (1, _M), 1) < length
    tx1_ref[0] = jnp.where(tm, tx1, 0.0)
    ty1_ref[0] = jnp.where(tm, ty1, 0.0)
    tx2_ref[0] = jnp.where(tm, tx1 + tw, 0.0)
    ty2_ref[0] = jnp.where(tm, ty1 + th, 0.0)
    tlab_ref[0] = jnp.where(tm, labi, -1)
    tsco_ref[0] = jnp.where(tm, 1.0, 0.0)
    tmsk_ref[0] = tm.astype(jnp.int32)


def _nms_body(key_ref, nbx1_ref, nby1_ref, nbx2_ref, nby2_ref,
              x1_ref, y1_ref, x2_ref, y2_ref, cls_ref, sco_ref,
              nbx1c_ref, nby1c_ref, nbx2c_ref, nby2c_ref,
              ox1_ref, oy1_ref, ox2_ref, oy2_ref, lab_ref, osco_ref,
              msk_ref, keep_ref):
    keyv = key_ref[0]
    valid = keyv < 0.0
    nv = jnp.sum(valid.astype(jnp.int32))
    nbx1 = nbx1_ref[0]
    nby1 = nby1_ref[0]
    nbx2 = nbx2_ref[0]
    nby2 = nby2_ref[0]
    area = (nbx2 - nbx1) * (nby2 - nby1)
    flat = (lax.broadcasted_iota(jnp.int32, (_ROWS, 128), 0) * 128
            + lax.broadcasted_iota(jnp.int32, (_ROWS, 128), 1))
    laneio = lax.broadcasted_iota(jnp.int32, (1, 128), 1)
    keep_ref[...] = valid.astype(jnp.int32)

    def body(i):
        r = i // 128
        c = i % 128
        bx1 = nbx1c_ref[0, i, 0]
        by1 = nby1c_ref[0, i, 0]
        bx2 = nbx2c_ref[0, i, 0]
        by2 = nby2c_ref[0, i, 0]
        krow = keep_ref[pl.ds(r, 1), :]
        ki = jnp.sum(jnp.where(laneio == c, krow, 0))
        a1 = (bx2 - bx1) * (by2 - by1)
        xx1 = jnp.maximum(bx1, nbx1)
        yy1 = jnp.maximum(by1, nby1)
        xx2 = jnp.minimum(bx2, nbx2)
        yy2 = jnp.minimum(by2, nby2)
        inter = (jnp.maximum(xx2 - xx1, 0.0) * jnp.maximum(yy2 - yy1, 0.0))
        iou = inter / (a1 + area - inter + 1e-9)
        sup = (iou > _NMS_T) & (flat > i) & (ki > 0)
        keep_ref[...] = jnp.where(sup, 0, keep_ref[...])
        return i + 1

    lax.while_loop(lambda i: i < nv, body, 0)

    kb = keep_ref[...] > 0
    ox1_ref[0] = jnp.where(kb, x1_ref[0], 0.0)
    oy1_ref[0] = jnp.where(kb, y1_ref[0], 0.0)
    ox2_ref[0] = jnp.where(kb, x2_ref[0], 0.0)
    oy2_ref[0] = jnp.where(kb, y2_ref[0], 0.0)
    lab_ref[0] = jnp.where(kb, cls_ref[0].astype(jnp.int32), -1)
    osco_ref[0] = jnp.where(kb, sco_ref[0], 0.0)
    msk_ref[0] = kb.astype(jnp.int32)


def _prep_call(pT, tT):
    f = jnp.float32
    i = jnp.int32
    vb = pl.BlockSpec((1, 1, _N), lambda b: (b, 0, 0))
    tb = pl.BlockSpec((1, 1, _M), lambda b: (b, 0, 0))
    return pl.pallas_call(
        _prep_body,
        grid=(_B,),
        in_specs=[pl.BlockSpec((1, 5 + _NUM_CLASSES, _N), lambda b: (b, 0, 0)),
                  pl.BlockSpec((1, 5, _M), lambda b: (b, 0, 0))],
        out_specs=[vb] * 11 + [tb] * 7,
        out_shape=([jax.ShapeDtypeStruct((_B, 1, _N), f)] * 11
                   + [jax.ShapeDtypeStruct((_B, 1, _M), f)] * 4
                   + [jax.ShapeDtypeStruct((_B, 1, _M), i),
                      jax.ShapeDtypeStruct((_B, 1, _M), f),
                      jax.ShapeDtypeStruct((_B, 1, _M), i)]),
    )(pT, tT)


def _nms_call(vecs, cols):
    f = jnp.float32
    i = jnp.int32
    vb = pl.BlockSpec((1, _ROWS, 128), lambda b: (b, 0, 0))
    cb = pl.BlockSpec((1, _NP, 1), lambda b: (b, 0, 0))
    return pl.pallas_call(
        _nms_body,
        grid=(_B,),
        in_specs=[vb] * 11 + [cb] * 4,
        out_specs=[vb] * 7,
        out_shape=([jax.ShapeDtypeStruct((_B, _ROWS, 128), f)] * 4
                   + [jax.ShapeDtypeStruct((_B, _ROWS, 128), i),
                      jax.ShapeDtypeStruct((_B, _ROWS, 128), f),
                      jax.ShapeDtypeStruct((_B, _ROWS, 128), i)]),
        scratch_shapes=[pltpu.VMEM((_ROWS, 128), i)],
    )(*vecs, *cols)


def kernel(predictions, targets):
    pT = predictions.transpose(0, 2, 1)
    tT = targets.transpose(0, 2, 1)
    outs = [a.reshape(a.shape[0], a.shape[2]) for a in _prep_call(pT, tT)]
    per_box = outs[:11]
    tx1, ty1, tx2, ty2, tlab, tsco, tmsk = outs[11:]

    pad = _NP - _N
    padded = [jnp.pad(per_box[0], ((0, 0), (0, pad)), constant_values=1.0)]
    padded += [jnp.pad(a, ((0, 0), (0, pad))) for a in per_box[1:]]
    s = lax.sort(padded, dimension=1, is_stable=True, num_keys=1)
    vecs = [a.reshape(_B, _ROWS, 128) for a in s]
    cols = [a.reshape(_B, _NP, 1) for a in s[1:5]]

    ox1, oy1, ox2, oy2, lab, osco, msk = _nms_call(vecs, cols)

    def unpack(a):
        return a.reshape(_B, _NP)[:, :_N]

    pred_boxes = jnp.stack([unpack(ox1), unpack(oy1),
                            unpack(ox2), unpack(oy2)], axis=-1)
    pred_labels = unpack(lab)
    pred_scores = unpack(osco)
    pred_mask = unpack(msk).astype(bool)
    tgt_boxes = jnp.stack([tx1, ty1, tx2, ty2], axis=-1)
    return (pred_boxes, pred_labels, pred_scores, pred_mask,
            tgt_boxes, tlab, tsco, tmsk.astype(bool))
